# Initial kernel scaffold; baseline (speedup 1.0000x reference)
import functools

import jax
import jax.numpy as jnp
from jax import lax
from jax.experimental import pallas as pl
from jax.experimental.pallas import tpu as pltpu
from jax.experimental.pallas import tpu_sc as plsc

_N = 100000
_CF = 32
_NW = 32
_ROWS_PER_W = _N // _NW
_CHUNK = 625
_NCHUNK = _ROWS_PER_W // _CHUNK


def _sc_upsample(feats3, coords, scalepat):
    mesh = plsc.VectorSubcoreMesh(core_axis_name="c", subcore_axis_name="s")

    @functools.partial(
        pl.kernel,
        mesh=mesh,
        out_type=(
            jax.ShapeDtypeStruct((_N, 8, _CF), jnp.float32),
            jax.ShapeDtypeStruct((_N * 32,), jnp.int32),
        ),
        scratch_types=[
            pltpu.VMEM((_CHUNK, 1, _CF), jnp.float32),
            pltpu.VMEM((_CHUNK, 4), jnp.int32),
            pltpu.VMEM((_CHUNK * 32,), jnp.int32),
            pltpu.VMEM((16,), jnp.int32),
            pltpu.SemaphoreType.DMA,
        ],
    )
    def k(feats_hbm, coords_hbm, scalepat_hbm, outf, outc, fbuf, cbuf, cexp,
          spat, sem):
        wid = lax.axis_index("s") * 2 + lax.axis_index("c")
        lane = lax.iota(jnp.int32, 16)
        c4 = lane & 3

        pltpu.sync_copy(scalepat_hbm, spat)
        sv = spat[...]

        def off_vec(j):
            return jnp.where(
                c4 == 1, (j >> 2) & 1,
                jnp.where(c4 == 2, (j >> 1) & 1,
                          jnp.where(c4 == 3, j & 1, 0)))

        off_a = off_vec(lane >> 2)
        off_b = off_vec((lane >> 2) + 4)

        row0 = wid * _ROWS_PER_W
        for ck in range(_NCHUNK):
            base = row0 + ck * _CHUNK
            pltpu.sync_copy(feats_hbm.at[pl.ds(base, _CHUNK)], fbuf)
            handles = [
                pltpu.async_copy(
                    fbuf, outf.at[pl.ds(base, _CHUNK), pl.ds(j, 1)], sem)
                for j in range(8)
            ]
            pltpu.sync_copy(coords_hbm.at[pl.ds(base, _CHUNK)], cbuf)

            def body(r, carry):
                g = plsc.load_gather(cbuf, [jnp.full((16,), r, jnp.int32), c4])
                scaled = g * sv
                cexp[pl.ds(r * 32, 16)] = scaled + off_a
                cexp[pl.ds(r * 32 + 16, 16)] = scaled + off_b
                return carry

            lax.fori_loop(0, _CHUNK, body, 0)
            pltpu.sync_copy(cexp, outc.at[pl.ds(base * 32, _CHUNK * 32)])
            for h in handles:
                h.wait()

    return k(feats3, coords, scalepat)


def kernel(feats, coords, scale_factor):
    sf = jnp.asarray(scale_factor, jnp.int32)
    one = jnp.ones((), jnp.int32)
    scalepat = jnp.tile(jnp.stack([one, sf, sf, sf]), 4)
    feats3 = feats.reshape(_N, 1, _CF)
    outf, outc = _sc_upsample(feats3, coords, scalepat)
    return outc.reshape(_N * 8, 4), outf.reshape(_N * 8, _CF)


# speedup vs baseline: 2.2973x; 2.2973x over previous
import functools

import jax
import jax.numpy as jnp
from jax import lax
from jax.experimental import pallas as pl
from jax.experimental.pallas import tpu as pltpu
from jax.experimental.pallas import tpu_sc as plsc

_N = 100000
_CF = 32
_NW = 32
_CH = 200
_NCHUNK = _N // _CH
_ITERS = -(-_NCHUNK // _NW)


def _sc_upsample(feats, coords, scalepat):
    mesh = plsc.VectorSubcoreMesh(core_axis_name="c", subcore_axis_name="s")

    @functools.partial(
        pl.kernel,
        mesh=mesh,
        out_type=(
            jax.ShapeDtypeStruct((_N, 8 * _CF), jnp.float32),
            jax.ShapeDtypeStruct((_N, 32), jnp.int32),
        ),
        scratch_types=[
            pltpu.VMEM((_CH, _CF), jnp.float32),
            pltpu.VMEM((_CH, 8 * _CF), jnp.float32),
            pltpu.VMEM((_CH, 4), jnp.int32),
            pltpu.VMEM((_CH, 32), jnp.int32),
            pltpu.VMEM((16,), jnp.int32),
        ],
        compiler_params=pltpu.CompilerParams(
            use_tc_tiling_on_sc=False,
            needs_layout_passes=False,
        ),
    )
    def k(feats_hbm, coords_hbm, scalepat_hbm, outf, outc, fbuf, fexp, cbuf,
          cexp, spat):
        wid = lax.axis_index("s") * 2 + lax.axis_index("c")
        lane = lax.iota(jnp.int32, 16)
        c4 = lane & 3

        pltpu.sync_copy(scalepat_hbm, spat)
        sv = spat[...]

        def off_vec(j):
            return jnp.where(
                c4 == 1, (j >> 2) & 1,
                jnp.where(c4 == 2, (j >> 1) & 1,
                          jnp.where(c4 == 3, j & 1, 0)))

        off_a = off_vec(lane >> 2)
        off_b = off_vec((lane >> 2) + 4)

        def chunk_body(t, carry):
            cid = t * _NW + wid

            @pl.when(cid < _NCHUNK)
            def _():
                base = pl.multiple_of(cid * _CH, _CH)
                pltpu.sync_copy(feats_hbm.at[pl.ds(base, _CH)], fbuf)
                pltpu.sync_copy(coords_hbm.at[pl.ds(base, _CH)], cbuf)

                def row_body(r, rcarry):
                    f0 = fbuf[r, pl.ds(0, 16)]
                    f1 = fbuf[r, pl.ds(16, 16)]
                    for j in range(8):
                        fexp[r, pl.ds(j * 32, 16)] = f0
                        fexp[r, pl.ds(j * 32 + 16, 16)] = f1
                    g = plsc.load_gather(
                        cbuf, [jnp.full((16,), r, jnp.int32), c4])
                    scaled = g * sv
                    cexp[r, pl.ds(0, 16)] = scaled + off_a
                    cexp[r, pl.ds(16, 16)] = scaled + off_b
                    return rcarry

                lax.fori_loop(0, _CH, row_body, 0)
                pltpu.sync_copy(fexp, outf.at[pl.ds(base, _CH)])
                pltpu.sync_copy(cexp, outc.at[pl.ds(base, _CH)])

            return carry

        lax.fori_loop(0, _ITERS, chunk_body, 0)

    return k(feats, coords, scalepat)


def kernel(feats, coords, scale_factor):
    sf = jnp.asarray(scale_factor, jnp.int32)
    one = jnp.ones((), jnp.int32)
    scalepat = jnp.tile(jnp.stack([one, sf, sf, sf]), 4)
    outf, outc = _sc_upsample(feats, coords, scalepat)
    return outc.reshape(_N * 8, 4), outf.reshape(_N * 8, _CF)


# speedup vs baseline: 13.3871x; 5.8273x over previous
import functools

import jax
import jax.numpy as jnp
from jax import lax
from jax.experimental import pallas as pl
from jax.experimental.pallas import tpu as pltpu
from jax.experimental.pallas import tpu_sc as plsc

_N = 100000
_CF = 32
_NW = 32
_NT = _N // 128
_TAIL = _N - _NT * 128
_ITERS = -(-_NT // _NW)


def _expand_row(src, dst, r, n_in_vregs, idx, scale=None, offs=None):
    for k in range(n_in_vregs):
        v = src[r, pl.ds(k * 16, 16)]
        for g in range(8):
            w = v.at[idx[g]].get(mode="promise_in_bounds")
            if scale is not None:
                w = w * scale
            if offs is not None:
                w = w + offs
            dst[r, pl.ds(k * 128 + g * 16, 16)] = w


def _sc_upsample(ft, ct, ftail, ctail, scalepat):
    mesh = plsc.VectorSubcoreMesh(core_axis_name="c", subcore_axis_name="s")

    @functools.partial(
        pl.kernel,
        mesh=mesh,
        out_type=(
            jax.ShapeDtypeStruct((_CF, 8 * _N), jnp.float32),
            jax.ShapeDtypeStruct((4, 8 * _N), jnp.int32),
        ),
        scratch_types=[
            pltpu.VMEM((_CF, 128), jnp.float32),
            pltpu.VMEM((_CF, 1024), jnp.float32),
            pltpu.VMEM((4, 128), jnp.int32),
            pltpu.VMEM((4, 1024), jnp.int32),
            pltpu.VMEM((16,), jnp.int32),
        ],
    )
    def k(ft_hbm, ct_hbm, ftail_hbm, ctail_hbm, sp_hbm, oft, oct,
          fbuf, fexp, cbuf, cexp, spat):
        wid = lax.axis_index("s") * 2 + lax.axis_index("c")
        lane = lax.iota(jnp.int32, 16)

        pltpu.sync_copy(sp_hbm, spat)
        sv = spat[...]

        idx = [(lane >> 3) + 2 * g for g in range(8)]
        j8 = lane & 7
        coffs = [None, (j8 >> 2) & 1, (j8 >> 1) & 1, j8 & 1]
        cscale = [None, sv, sv, sv]

        def expand_chunk(n_in_vregs):
            for r in range(_CF):
                _expand_row(fbuf, fexp, r, n_in_vregs, idx)
            for r in range(4):
                _expand_row(cbuf, cexp, r, n_in_vregs, idx,
                            scale=cscale[r], offs=coffs[r])

        def chunk_body(t, carry):
            cid = t * _NW + wid

            @pl.when(cid < _NT)
            def _():
                src = pl.multiple_of(cid * 128, 128)
                dst = pl.multiple_of(cid * 1024, 128)
                pltpu.sync_copy(ft_hbm.at[:, pl.ds(src, 128)], fbuf)
                pltpu.sync_copy(ct_hbm.at[:, pl.ds(src, 128)], cbuf)
                expand_chunk(8)
                pltpu.sync_copy(fexp, oft.at[:, pl.ds(dst, 1024)])
                pltpu.sync_copy(cexp, oct.at[:, pl.ds(dst, 1024)])

            return carry

        lax.fori_loop(0, _ITERS, chunk_body, 0)

        @pl.when(wid == _NW - 1)
        def _tail():
            dst = _NT * 1024
            pltpu.sync_copy(ftail_hbm, fbuf)
            pltpu.sync_copy(ctail_hbm, cbuf)
            expand_chunk(_TAIL // 16)
            pltpu.sync_copy(
                fexp.at[:, pl.ds(0, 8 * _TAIL)],
                oft.at[:, pl.ds(dst, 8 * _TAIL)])
            pltpu.sync_copy(
                cexp.at[:, pl.ds(0, 8 * _TAIL)],
                oct.at[:, pl.ds(dst, 8 * _TAIL)])

    return k(ft, ct, ftail, ctail, scalepat)


def kernel(feats, coords, scale_factor):
    sf = jnp.asarray(scale_factor, jnp.int32)
    scalepat = jnp.full((16,), sf, jnp.int32)
    pad = ((0, 0), (0, 128 - _TAIL))
    ftail = jnp.pad(feats[_NT * 128:].T, pad)
    ctail = jnp.pad(coords[_NT * 128:].T, pad)
    oft, oct = _sc_upsample(feats.T, coords.T, ftail, ctail, scalepat)
    return oct.T, oft.T
